# baseline (speedup 1.0000x reference)
import functools

import jax
import jax.numpy as jnp
from jax import lax
from jax.experimental import pallas as pl
from jax.experimental.pallas import tpu as pltpu
from jax.experimental.pallas import tpu_sc as plsc

NC = 2
NS = 16
L = 16
NW = NC * NS
NBUF = 5
PREF = 3


def _tc_linear(X, W, b):
    N, D = X.shape
    H = W.shape[0]
    BR = 1000

    def body(x_ref, w_ref, b_ref, o_ref):
        o_ref[...] = lax.dot_general(
            x_ref[...], w_ref[...], (((1,), (1,)), ((), ())),
            preferred_element_type=jnp.float32) + b_ref[...]

    return pl.pallas_call(
        body,
        grid=(N // BR,),
        in_specs=[
            pl.BlockSpec((BR, D), lambda i: (i, 0)),
            pl.BlockSpec((H, D), lambda i: (0, 0)),
            pl.BlockSpec((1, H), lambda i: (0, 0)),
        ],
        out_specs=pl.BlockSpec((BR, H), lambda i: (i, 0)),
        out_shape=jax.ShapeDtypeStruct((N, H), jnp.float32),
    )(X, W, b.reshape(1, H))


def _tc_combine_act_linear(hp, W, b):
    _, N, H = hp.shape
    Z = W.shape[0]
    BR = 1000

    def body(h_ref, w_ref, b_ref, o_ref):
        h = h_ref[0] + h_ref[1]
        h = jnp.where(h >= 0, h, 0.1 * h)
        o_ref[...] = lax.dot_general(
            h, w_ref[...], (((1,), (1,)), ((), ())),
            preferred_element_type=jnp.float32) + b_ref[...]

    return pl.pallas_call(
        body,
        grid=(N // BR,),
        in_specs=[
            pl.BlockSpec((2, BR, H), lambda i: (0, i, 0)),
            pl.BlockSpec((Z, H), lambda i: (0, 0)),
            pl.BlockSpec((1, Z), lambda i: (0, 0)),
        ],
        out_specs=pl.BlockSpec((BR, Z), lambda i: (i, 0)),
        out_shape=jax.ShapeDtypeStruct((N, Z), jnp.float32),
    )(hp, W, b.reshape(1, Z))


def _tc_combine_clean(zp):
    _, N, Z = zp.shape
    BR = 1000

    def body(z_ref, o_ref):
        z = z_ref[0] + z_ref[1]
        o_ref[...] = jnp.where(jnp.isfinite(z), z, 0.0)

    return pl.pallas_call(
        body,
        grid=(N // BR,),
        in_specs=[pl.BlockSpec((2, BR, Z), lambda i: (0, i, 0))],
        out_specs=pl.BlockSpec((BR, Z), lambda i: (i, 0)),
        out_shape=jax.ShapeDtypeStruct((N, Z), jnp.float32),
    )(zp)


def _sc_spmm(T, row2d, col2d, val2d, zeros, *, C=80, SBUF=2):
    N, D = T.shape
    E = row2d.shape[0] * C
    Et = E // NW
    nchunk = Et // C
    rpt = ((N // NS + 7) // 8) * 8
    rpt_last = N - (NS - 1) * rpt

    mesh = plsc.VectorSubcoreMesh(core_axis_name="c", subcore_axis_name="s",
                                  num_cores=NC, num_subcores=NS)

    @functools.partial(
        pl.kernel,
        out_type=jax.ShapeDtypeStruct((NC, N, D), jnp.float32),
        mesh=mesh,
        compiler_params=pltpu.CompilerParams(use_tc_tiling_on_sc=False,
                                             needs_layout_passes=False),
        scratch_types=(
            [pltpu.VMEM_SHARED((N, D), jnp.float32)]
            + [pltpu.VMEM((nchunk, C), jnp.int32)] * 2
            + [pltpu.VMEM((nchunk, C), jnp.float32)]
            + [pltpu.VMEM((C,), jnp.int32)]
            + [pltpu.VMEM((C, D), jnp.float32)] * SBUF
            + [pltpu.SemaphoreType.DMA] * SBUF
        ),
    )
    def spmm(t_hbm, row_hbm, col_hbm, val_hbm, zero_hbm, out_hbm,
             acc_sh, rowv, colv, valv, rowtmp, *rest):
        bufs = rest[:SBUF]
        sem_g = rest[SBUF:2 * SBUF]
        c = lax.axis_index("c")
        s = lax.axis_index("s")
        wid = c * NS + s
        r0 = s * rpt
        k0 = wid * nchunk

        @pl.when(s < NS - 1)
        def _():
            pltpu.sync_copy(zero_hbm.at[pl.ds(r0, rpt)],
                            acc_sh.at[pl.ds(r0, rpt)])

        @pl.when(s == NS - 1)
        def _():
            pltpu.sync_copy(zero_hbm.at[pl.ds((NS - 1) * rpt, rpt_last)],
                            acc_sh.at[pl.ds((NS - 1) * rpt, rpt_last)])

        pltpu.sync_copy(col_hbm.at[pl.ds(k0, nchunk)], colv)
        pltpu.sync_copy(row_hbm.at[pl.ds(k0, nchunk)], rowv)
        pltpu.sync_copy(val_hbm.at[pl.ds(k0, nchunk)], valv)
        plsc.subcore_barrier()

        def gather(k, b):
            pltpu.async_copy(t_hbm.at[colv.at[k]], bufs[b], sem_g[b])

        def process(k, b):
            pltpu.make_async_copy(t_hbm.at[colv.at[k]], bufs[b],
                                  sem_g[b]).wait()

            def scale_body(gg, carry2):
                vv = valv[k, pl.ds(gg * L, L)]
                for e2 in range(L):
                    v = lax.gather(
                        vv, jnp.full((L, 1), e2, jnp.int32),
                        lax.GatherDimensionNumbers(
                            offset_dims=(), collapsed_slice_dims=(0,),
                            start_index_map=(0,)),
                        (1,), mode=lax.GatherScatterMode.PROMISE_IN_BOUNDS)
                    e = gg * L + e2
                    for j in range(D // L):
                        bufs[b][e, pl.ds(j * L, L)] = (
                            bufs[b][e, pl.ds(j * L, L)] * v)
                return carry2

            lax.fori_loop(0, C // L, scale_body, 0)
            for j in range(C // L):
                rowtmp[pl.ds(j * L, L)] = rowv[k, pl.ds(j * L, L)]
            pltpu.sync_copy(bufs[b], acc_sh.at[rowtmp], add=True)
            gather(jnp.minimum(k + SBUF, nchunk - 1), b)

        for b in range(SBUF):
            gather(b, b)
        process(0, 0)

        def outer_body(m, carry):
            for j in range(SBUF):
                k = m * SBUF + 1 + j
                process(k, (1 + j) % SBUF)
            return carry

        lax.fori_loop(0, (nchunk - 1) // SBUF, outer_body, 0)
        for b in range(SBUF):
            pltpu.make_async_copy(t_hbm.at[colv.at[nchunk - 1]], bufs[b],
                                  sem_g[b]).wait()
        plsc.subcore_barrier()

        @pl.when(s < NS - 1)
        def _():
            pltpu.sync_copy(acc_sh.at[pl.ds(r0, rpt)],
                            out_hbm.at[c, pl.ds(r0, rpt)])

        @pl.when(s == NS - 1)
        def _():
            pltpu.sync_copy(acc_sh.at[pl.ds((NS - 1) * rpt, rpt_last)],
                            out_hbm.at[c, pl.ds((NS - 1) * rpt, rpt_last)])

    return spmm(T, row2d, col2d, val2d, zeros)


def _sc_decode(z, src2d, dst2d, bias16, *, C=80):
    N, Z = z.shape
    E = src2d.shape[0] * C
    Et = E // NW
    nchunk = Et // C

    mesh = plsc.VectorSubcoreMesh(core_axis_name="c", subcore_axis_name="s",
                                  num_cores=NC, num_subcores=NS)

    @functools.partial(
        pl.kernel,
        out_type=jax.ShapeDtypeStruct((E // C, C), jnp.float32),
        mesh=mesh,
        compiler_params=pltpu.CompilerParams(use_tc_tiling_on_sc=False,
                                             needs_layout_passes=False),
        scratch_types=(
            [pltpu.VMEM((nchunk, C), jnp.int32)] * 2
            + [pltpu.VMEM((nchunk, C), jnp.float32)]
            + [pltpu.VMEM((L,), jnp.float32)]
            + [pltpu.VMEM((L * L,), jnp.float32)]
            + [pltpu.VMEM((C, Z), jnp.float32)] * (2 * NBUF)
            + [pltpu.SemaphoreType.DMA] * (2 * NBUF)
        ),
    )
    def dec(z_hbm, src_hbm, dst_hbm, bias_hbm, out_hbm,
            srcv, dstv, obig, bvm, tbuf, *bufs_sems):
        zsb = bufs_sems[:NBUF]
        zdb = bufs_sems[NBUF:2 * NBUF]
        sem_a = bufs_sems[2 * NBUF:3 * NBUF]
        sem_b = bufs_sems[3 * NBUF:]
        c = lax.axis_index("c")
        s = lax.axis_index("s")
        wid = c * NS + s
        k0 = wid * nchunk

        pltpu.sync_copy(bias_hbm, bvm)
        b0 = bvm[pl.ds(0, L)][0]
        lanes = lax.iota(jnp.int32, L)
        pltpu.sync_copy(src_hbm.at[pl.ds(k0, nchunk)], srcv)
        pltpu.sync_copy(dst_hbm.at[pl.ds(k0, nchunk)], dstv)

        def gather(k, b):
            pltpu.async_copy(z_hbm.at[srcv.at[k]], zsb[b], sem_a[b])
            pltpu.async_copy(z_hbm.at[dstv.at[k]], zdb[b], sem_b[b])

        for b in range(NBUF):
            gather(b, b)

        def outer_body(g, carry):
            for b in range(NBUF):
                k = g * NBUF + b
                pltpu.make_async_copy(z_hbm.at[srcv.at[k]], zsb[b],
                                      sem_a[b]).wait()
                pltpu.make_async_copy(z_hbm.at[dstv.at[k]], zdb[b],
                                      sem_b[b]).wait()

                def group_body(gg, carry2):
                    for e2 in range(L):
                        e = gg * L + e2
                        acc = zsb[b][e, pl.ds(0, L)] * zdb[b][e, pl.ds(0, L)]
                        for j in range(1, Z // L):
                            acc = acc + (zsb[b][e, pl.ds(j * L, L)]
                                         * zdb[b][e, pl.ds(j * L, L)])
                        plsc.store_scatter(tbuf, [lanes * L + e2], acc)
                    tot = tbuf[pl.ds(0, L)]
                    for l in range(1, L):
                        tot = tot + tbuf[pl.ds(l * L, L)]
                    v = tot + b0
                    v = jnp.where(v != v, 0.0, v)
                    v = jnp.minimum(jnp.maximum(v, -20.0), 20.0)
                    obig[k, pl.ds(gg * L, L)] = v
                    return carry2

                lax.fori_loop(0, C // L, group_body, 0)
                gather(jnp.minimum(k + NBUF, nchunk - 1), b)
            return carry

        lax.fori_loop(0, nchunk // NBUF, outer_body, 0)
        for b in range(NBUF):
            pltpu.make_async_copy(z_hbm.at[srcv.at[nchunk - 1]], zsb[b],
                                  sem_a[b]).wait()
            pltpu.make_async_copy(z_hbm.at[dstv.at[nchunk - 1]], zdb[b],
                                  sem_b[b]).wait()
        pltpu.sync_copy(obig, out_hbm.at[pl.ds(k0, nchunk)])

    return dec(z, src2d, dst2d, bias16)


def kernel(X, adj_indices, adj_values, edge_index, W1, b1, W2, b2, dec_bias):
    N, D = X.shape
    H = W1.shape[0]
    Z = W2.shape[0]
    E = adj_values.shape[0]
    CS = 80
    CD = 80

    Ep = ((E + NW * CS - 1) // (NW * CS)) * (NW * CS)
    pad = Ep - E
    if pad:
        row = jnp.concatenate([adj_indices[0],
                               jnp.zeros((pad,), adj_indices.dtype)])
        col = jnp.concatenate([adj_indices[1],
                               jnp.zeros((pad,), adj_indices.dtype)])
        val = jnp.concatenate([adj_values,
                               jnp.zeros((pad,), adj_values.dtype)])
    else:
        row, col, val = adj_indices[0], adj_indices[1], adj_values
    row = row.reshape(Ep // CS, CS)
    col = col.reshape(Ep // CS, CS)
    val = val.reshape(Ep // CS, CS)

    xw = _tc_linear(X, W1, b1)
    hp = _sc_spmm(xw, row, col, val, jnp.zeros((N, H), jnp.float32),
                  C=CS, SBUF=2)
    hw = _tc_combine_act_linear(hp, W2, b2)
    zp = _sc_spmm(hw, row, col, val, jnp.zeros((N, Z), jnp.float32),
                  C=CS, SBUF=4)
    z = _tc_combine_clean(zp)
    bias16 = jnp.broadcast_to(dec_bias, (L,)).astype(jnp.float32)
    logits = _sc_decode(z, edge_index[0].reshape(E // CD, CD),
                        edge_index[1].reshape(E // CD, CD), bias16, C=CD)
    return logits.reshape(E)


# speedup vs baseline: 1.4109x; 1.4109x over previous
import functools

import jax
import jax.numpy as jnp
from jax import lax
from jax.experimental import pallas as pl
from jax.experimental.pallas import tpu as pltpu
from jax.experimental.pallas import tpu_sc as plsc

NC = 2
NS = 16
L = 16
NW = NC * NS
NBUF = 5
PREF = 3


def _tc_linear(X, W, b):
    N, D = X.shape
    H = W.shape[0]
    BR = 1000

    def body(x_ref, w_ref, b_ref, o_ref):
        o_ref[...] = lax.dot_general(
            x_ref[...], w_ref[...], (((1,), (1,)), ((), ())),
            preferred_element_type=jnp.float32) + b_ref[...]

    return pl.pallas_call(
        body,
        grid=(N // BR,),
        in_specs=[
            pl.BlockSpec((BR, D), lambda i: (i, 0)),
            pl.BlockSpec((H, D), lambda i: (0, 0)),
            pl.BlockSpec((1, H), lambda i: (0, 0)),
        ],
        out_specs=pl.BlockSpec((BR, H), lambda i: (i, 0)),
        out_shape=jax.ShapeDtypeStruct((N, H), jnp.float32),
    )(X, W, b.reshape(1, H))


def _tc_combine_act_linear(hp, W, b):
    _, N, H = hp.shape
    Z = W.shape[0]
    BR = 1000

    def body(h_ref, w_ref, b_ref, o_ref):
        h = h_ref[0] + h_ref[1]
        h = jnp.where(h >= 0, h, 0.1 * h)
        o_ref[...] = lax.dot_general(
            h, w_ref[...], (((1,), (1,)), ((), ())),
            preferred_element_type=jnp.float32) + b_ref[...]

    return pl.pallas_call(
        body,
        grid=(N // BR,),
        in_specs=[
            pl.BlockSpec((2, BR, H), lambda i: (0, i, 0)),
            pl.BlockSpec((Z, H), lambda i: (0, 0)),
            pl.BlockSpec((1, Z), lambda i: (0, 0)),
        ],
        out_specs=pl.BlockSpec((BR, Z), lambda i: (i, 0)),
        out_shape=jax.ShapeDtypeStruct((N, Z), jnp.float32),
    )(hp, W, b.reshape(1, Z))


def _tc_combine_clean(zp):
    _, N, Z = zp.shape
    BR = 1000

    def body(z_ref, o_ref):
        z = z_ref[0] + z_ref[1]
        o_ref[...] = jnp.where(jnp.isfinite(z), z, 0.0)

    return pl.pallas_call(
        body,
        grid=(N // BR,),
        in_specs=[pl.BlockSpec((2, BR, Z), lambda i: (0, i, 0))],
        out_specs=pl.BlockSpec((BR, Z), lambda i: (i, 0)),
        out_shape=jax.ShapeDtypeStruct((N, Z), jnp.float32),
    )(zp)


def _sc_spmm(T, row2d, col2d, val2d, zeros, *, C=80, SBUF=2):
    N, D = T.shape
    E = row2d.shape[0] * C
    Et = E // NW
    nchunk = Et // C
    rpt = ((N // NS + 7) // 8) * 8
    rpt_last = N - (NS - 1) * rpt

    mesh = plsc.VectorSubcoreMesh(core_axis_name="c", subcore_axis_name="s",
                                  num_cores=NC, num_subcores=NS)

    @functools.partial(
        pl.kernel,
        out_type=jax.ShapeDtypeStruct((NC, N, D), jnp.float32),
        mesh=mesh,
        compiler_params=pltpu.CompilerParams(use_tc_tiling_on_sc=False,
                                             needs_layout_passes=False),
        scratch_types=(
            [pltpu.VMEM_SHARED((N, D), jnp.float32)]
            + [pltpu.VMEM((nchunk, C), jnp.int32)] * 2
            + [pltpu.VMEM((nchunk, C), jnp.float32)]
            + [pltpu.VMEM((C,), jnp.int32)]
            + [pltpu.VMEM((C, D), jnp.float32)] * SBUF
            + [pltpu.SemaphoreType.DMA] * SBUF
        ),
    )
    def spmm(t_hbm, row_hbm, col_hbm, val_hbm, zero_hbm, out_hbm,
             acc_sh, rowv, colv, valv, rowtmp, *rest):
        bufs = rest[:SBUF]
        sem_g = rest[SBUF:2 * SBUF]
        c = lax.axis_index("c")
        s = lax.axis_index("s")
        wid = c * NS + s
        r0 = s * rpt
        k0 = wid * nchunk

        @pl.when(s < NS - 1)
        def _():
            pltpu.sync_copy(zero_hbm.at[pl.ds(r0, rpt)],
                            acc_sh.at[pl.ds(r0, rpt)])

        @pl.when(s == NS - 1)
        def _():
            pltpu.sync_copy(zero_hbm.at[pl.ds((NS - 1) * rpt, rpt_last)],
                            acc_sh.at[pl.ds((NS - 1) * rpt, rpt_last)])

        pltpu.sync_copy(col_hbm.at[pl.ds(k0, nchunk)], colv)
        pltpu.sync_copy(row_hbm.at[pl.ds(k0, nchunk)], rowv)
        pltpu.sync_copy(val_hbm.at[pl.ds(k0, nchunk)], valv)
        plsc.subcore_barrier()

        def gather(k, b):
            pltpu.async_copy(t_hbm.at[colv.at[k]], bufs[b], sem_g[b])

        def process(k, b):
            pltpu.make_async_copy(t_hbm.at[colv.at[k]], bufs[b],
                                  sem_g[b]).wait()

            @functools.partial(plsc.parallel_loop, 0, C // L)
            def scale_body(gg):
                vv = valv[k, pl.ds(gg * L, L)]
                for e2 in range(L):
                    v = lax.gather(
                        vv, jnp.full((L, 1), e2, jnp.int32),
                        lax.GatherDimensionNumbers(
                            offset_dims=(), collapsed_slice_dims=(0,),
                            start_index_map=(0,)),
                        (1,), mode=lax.GatherScatterMode.PROMISE_IN_BOUNDS)
                    e = gg * L + e2
                    for j in range(D // L):
                        bufs[b][e, pl.ds(j * L, L)] = (
                            bufs[b][e, pl.ds(j * L, L)] * v)
            for j in range(C // L):
                rowtmp[pl.ds(j * L, L)] = rowv[k, pl.ds(j * L, L)]
            pltpu.sync_copy(bufs[b], acc_sh.at[rowtmp], add=True)
            gather(jnp.minimum(k + SBUF, nchunk - 1), b)

        for b in range(SBUF):
            gather(b, b)
        process(0, 0)

        def outer_body(m, carry):
            for j in range(SBUF):
                k = m * SBUF + 1 + j
                process(k, (1 + j) % SBUF)
            return carry

        lax.fori_loop(0, (nchunk - 1) // SBUF, outer_body, 0)
        for b in range(SBUF):
            pltpu.make_async_copy(t_hbm.at[colv.at[nchunk - 1]], bufs[b],
                                  sem_g[b]).wait()
        plsc.subcore_barrier()

        @pl.when(s < NS - 1)
        def _():
            pltpu.sync_copy(acc_sh.at[pl.ds(r0, rpt)],
                            out_hbm.at[c, pl.ds(r0, rpt)])

        @pl.when(s == NS - 1)
        def _():
            pltpu.sync_copy(acc_sh.at[pl.ds((NS - 1) * rpt, rpt_last)],
                            out_hbm.at[c, pl.ds((NS - 1) * rpt, rpt_last)])

    return spmm(T, row2d, col2d, val2d, zeros)


def _sc_decode(z, src2d, dst2d, bias16, *, C=80):
    N, Z = z.shape
    E = src2d.shape[0] * C
    Et = E // NW
    nchunk = Et // C

    mesh = plsc.VectorSubcoreMesh(core_axis_name="c", subcore_axis_name="s",
                                  num_cores=NC, num_subcores=NS)

    @functools.partial(
        pl.kernel,
        out_type=jax.ShapeDtypeStruct((E // C, C), jnp.float32),
        mesh=mesh,
        compiler_params=pltpu.CompilerParams(use_tc_tiling_on_sc=False,
                                             needs_layout_passes=False),
        scratch_types=(
            [pltpu.VMEM((nchunk, C), jnp.int32)] * 2
            + [pltpu.VMEM((nchunk, C), jnp.float32)]
            + [pltpu.VMEM((L,), jnp.float32)]
            + [pltpu.VMEM((L * L,), jnp.float32)]
            + [pltpu.VMEM((C, Z), jnp.float32)] * (2 * NBUF)
            + [pltpu.SemaphoreType.DMA] * (2 * NBUF)
        ),
    )
    def dec(z_hbm, src_hbm, dst_hbm, bias_hbm, out_hbm,
            srcv, dstv, obig, bvm, tbuf, *bufs_sems):
        zsb = bufs_sems[:NBUF]
        zdb = bufs_sems[NBUF:2 * NBUF]
        sem_a = bufs_sems[2 * NBUF:3 * NBUF]
        sem_b = bufs_sems[3 * NBUF:]
        c = lax.axis_index("c")
        s = lax.axis_index("s")
        wid = c * NS + s
        k0 = wid * nchunk

        pltpu.sync_copy(bias_hbm, bvm)
        b0 = bvm[pl.ds(0, L)][0]
        lanes = lax.iota(jnp.int32, L)
        pltpu.sync_copy(src_hbm.at[pl.ds(k0, nchunk)], srcv)
        pltpu.sync_copy(dst_hbm.at[pl.ds(k0, nchunk)], dstv)

        def gather(k, b):
            pltpu.async_copy(z_hbm.at[srcv.at[k]], zsb[b], sem_a[b])
            pltpu.async_copy(z_hbm.at[dstv.at[k]], zdb[b], sem_b[b])

        for b in range(NBUF):
            gather(b, b)

        def outer_body(g, carry):
            for b in range(NBUF):
                k = g * NBUF + b
                pltpu.make_async_copy(z_hbm.at[srcv.at[k]], zsb[b],
                                      sem_a[b]).wait()
                pltpu.make_async_copy(z_hbm.at[dstv.at[k]], zdb[b],
                                      sem_b[b]).wait()

                def group_body(gg, carry2):
                    for e2 in range(L):
                        e = gg * L + e2
                        acc = zsb[b][e, pl.ds(0, L)] * zdb[b][e, pl.ds(0, L)]
                        for j in range(1, Z // L):
                            acc = acc + (zsb[b][e, pl.ds(j * L, L)]
                                         * zdb[b][e, pl.ds(j * L, L)])
                        plsc.store_scatter(tbuf, [lanes * L + e2], acc)
                    tot = tbuf[pl.ds(0, L)]
                    for l in range(1, L):
                        tot = tot + tbuf[pl.ds(l * L, L)]
                    v = tot + b0
                    v = jnp.where(v != v, 0.0, v)
                    v = jnp.minimum(jnp.maximum(v, -20.0), 20.0)
                    obig[k, pl.ds(gg * L, L)] = v
                    return carry2

                lax.fori_loop(0, C // L, group_body, 0)
                gather(jnp.minimum(k + NBUF, nchunk - 1), b)
            return carry

        lax.fori_loop(0, nchunk // NBUF, outer_body, 0)
        for b in range(NBUF):
            pltpu.make_async_copy(z_hbm.at[srcv.at[nchunk - 1]], zsb[b],
                                  sem_a[b]).wait()
            pltpu.make_async_copy(z_hbm.at[dstv.at[nchunk - 1]], zdb[b],
                                  sem_b[b]).wait()
        pltpu.sync_copy(obig, out_hbm.at[pl.ds(k0, nchunk)])

    return dec(z, src2d, dst2d, bias16)


def kernel(X, adj_indices, adj_values, edge_index, W1, b1, W2, b2, dec_bias):
    N, D = X.shape
    H = W1.shape[0]
    Z = W2.shape[0]
    E = adj_values.shape[0]
    CS = 80
    CD = 80

    Ep = ((E + NW * CS - 1) // (NW * CS)) * (NW * CS)
    pad = Ep - E
    if pad:
        row = jnp.concatenate([adj_indices[0],
                               jnp.zeros((pad,), adj_indices.dtype)])
        col = jnp.concatenate([adj_indices[1],
                               jnp.zeros((pad,), adj_indices.dtype)])
        val = jnp.concatenate([adj_values,
                               jnp.zeros((pad,), adj_values.dtype)])
    else:
        row, col, val = adj_indices[0], adj_indices[1], adj_values
    row = row.reshape(Ep // CS, CS)
    col = col.reshape(Ep // CS, CS)
    val = val.reshape(Ep // CS, CS)

    xw = _tc_linear(X, W1, b1)
    hp = _sc_spmm(xw, row, col, val, jnp.zeros((N, H), jnp.float32),
                  C=CS, SBUF=2)
    hw = _tc_combine_act_linear(hp, W2, b2)
    zp = _sc_spmm(hw, row, col, val, jnp.zeros((N, Z), jnp.float32),
                  C=CS, SBUF=4)
    z = _tc_combine_clean(zp)
    bias16 = jnp.broadcast_to(dec_bias, (L,)).astype(jnp.float32)
    logits = _sc_decode(z, edge_index[0].reshape(E // CD, CD),
                        edge_index[1].reshape(E // CD, CD), bias16, C=CD)
    return logits.reshape(E)


# speedup vs baseline: 1.4126x; 1.0012x over previous
import functools

import jax
import jax.numpy as jnp
from jax import lax
from jax.experimental import pallas as pl
from jax.experimental.pallas import tpu as pltpu
from jax.experimental.pallas import tpu_sc as plsc

NC = 2
NS = 16
L = 16
NW = NC * NS
NBUF = 5
PREF = 3


def _tc_linear(X, W, b):
    N, D = X.shape
    H = W.shape[0]
    BR = 1000

    def body(x_ref, w_ref, b_ref, o_ref):
        o_ref[...] = lax.dot_general(
            x_ref[...], w_ref[...], (((1,), (1,)), ((), ())),
            preferred_element_type=jnp.float32) + b_ref[...]

    return pl.pallas_call(
        body,
        grid=(N // BR,),
        in_specs=[
            pl.BlockSpec((BR, D), lambda i: (i, 0)),
            pl.BlockSpec((H, D), lambda i: (0, 0)),
            pl.BlockSpec((1, H), lambda i: (0, 0)),
        ],
        out_specs=pl.BlockSpec((BR, H), lambda i: (i, 0)),
        out_shape=jax.ShapeDtypeStruct((N, H), jnp.float32),
    )(X, W, b.reshape(1, H))


def _tc_combine_act_linear(hp, W, b):
    _, N, H = hp.shape
    Z = W.shape[0]
    BR = 1000

    def body(h_ref, w_ref, b_ref, o_ref):
        h = h_ref[0] + h_ref[1]
        h = jnp.where(h >= 0, h, 0.1 * h)
        o_ref[...] = lax.dot_general(
            h, w_ref[...], (((1,), (1,)), ((), ())),
            preferred_element_type=jnp.float32) + b_ref[...]

    return pl.pallas_call(
        body,
        grid=(N // BR,),
        in_specs=[
            pl.BlockSpec((2, BR, H), lambda i: (0, i, 0)),
            pl.BlockSpec((Z, H), lambda i: (0, 0)),
            pl.BlockSpec((1, Z), lambda i: (0, 0)),
        ],
        out_specs=pl.BlockSpec((BR, Z), lambda i: (i, 0)),
        out_shape=jax.ShapeDtypeStruct((N, Z), jnp.float32),
    )(hp, W, b.reshape(1, Z))


def _tc_combine_clean(zp):
    _, N, Z = zp.shape
    BR = 1000

    def body(z_ref, o_ref):
        z = z_ref[0] + z_ref[1]
        o_ref[...] = jnp.where(jnp.isfinite(z), z, 0.0)

    return pl.pallas_call(
        body,
        grid=(N // BR,),
        in_specs=[pl.BlockSpec((2, BR, Z), lambda i: (0, i, 0))],
        out_specs=pl.BlockSpec((BR, Z), lambda i: (i, 0)),
        out_shape=jax.ShapeDtypeStruct((N, Z), jnp.float32),
    )(zp)


def _sc_spmm(T, row2d, col2d, val2d, zeros, *, C=80, SBUF=2):
    N, D = T.shape
    E = row2d.shape[0] * C
    Et = E // NW
    nchunk = Et // C
    rpt = ((N // NS + 7) // 8) * 8
    rpt_last = N - (NS - 1) * rpt

    mesh = plsc.VectorSubcoreMesh(core_axis_name="c", subcore_axis_name="s",
                                  num_cores=NC, num_subcores=NS)

    @functools.partial(
        pl.kernel,
        out_type=jax.ShapeDtypeStruct((NC, N, D), jnp.float32),
        mesh=mesh,
        compiler_params=pltpu.CompilerParams(use_tc_tiling_on_sc=False,
                                             needs_layout_passes=False),
        scratch_types=(
            [pltpu.VMEM_SHARED((N, D), jnp.float32)]
            + [pltpu.VMEM((nchunk, C), jnp.int32)] * 2
            + [pltpu.VMEM((nchunk, C), jnp.float32)]
            + [pltpu.VMEM((C,), jnp.int32)]
            + [pltpu.VMEM((C, D), jnp.float32)] * SBUF
            + [pltpu.SemaphoreType.DMA] * SBUF
        ),
    )
    def spmm(t_hbm, row_hbm, col_hbm, val_hbm, zero_hbm, out_hbm,
             acc_sh, rowv, colv, valv, rowtmp, *rest):
        bufs = rest[:SBUF]
        sem_g = rest[SBUF:2 * SBUF]
        c = lax.axis_index("c")
        s = lax.axis_index("s")
        wid = c * NS + s
        r0 = s * rpt
        k0 = wid * nchunk

        @pl.when(s < NS - 1)
        def _():
            pltpu.sync_copy(zero_hbm.at[pl.ds(r0, rpt)],
                            acc_sh.at[pl.ds(r0, rpt)])

        @pl.when(s == NS - 1)
        def _():
            pltpu.sync_copy(zero_hbm.at[pl.ds((NS - 1) * rpt, rpt_last)],
                            acc_sh.at[pl.ds((NS - 1) * rpt, rpt_last)])

        pltpu.sync_copy(col_hbm.at[pl.ds(k0, nchunk)], colv)
        pltpu.sync_copy(row_hbm.at[pl.ds(k0, nchunk)], rowv)
        pltpu.sync_copy(val_hbm.at[pl.ds(k0, nchunk)], valv)
        plsc.subcore_barrier()

        def gather(k, b):
            pltpu.async_copy(t_hbm.at[colv.at[k]], bufs[b], sem_g[b])

        def process(k, b):
            pltpu.make_async_copy(t_hbm.at[colv.at[k]], bufs[b],
                                  sem_g[b]).wait()

            @functools.partial(plsc.parallel_loop, 0, C // L)
            def scale_body(gg):
                vv = valv[k, pl.ds(gg * L, L)]
                for e2 in range(L):
                    v = lax.gather(
                        vv, jnp.full((L, 1), e2, jnp.int32),
                        lax.GatherDimensionNumbers(
                            offset_dims=(), collapsed_slice_dims=(0,),
                            start_index_map=(0,)),
                        (1,), mode=lax.GatherScatterMode.PROMISE_IN_BOUNDS)
                    e = gg * L + e2
                    for j in range(D // L):
                        bufs[b][e, pl.ds(j * L, L)] = (
                            bufs[b][e, pl.ds(j * L, L)] * v)
            for j in range(C // L):
                rowtmp[pl.ds(j * L, L)] = rowv[k, pl.ds(j * L, L)]
            pltpu.sync_copy(bufs[b], acc_sh.at[rowtmp], add=True)
            gather(jnp.minimum(k + SBUF, nchunk - 1), b)

        for b in range(SBUF):
            gather(b, b)
        process(0, 0)

        def outer_body(m, carry):
            for j in range(SBUF):
                k = m * SBUF + 1 + j
                process(k, (1 + j) % SBUF)
            return carry

        lax.fori_loop(0, (nchunk - 1) // SBUF, outer_body, 0)
        for b in range(SBUF):
            pltpu.make_async_copy(t_hbm.at[colv.at[nchunk - 1]], bufs[b],
                                  sem_g[b]).wait()
        plsc.subcore_barrier()

        @pl.when(s < NS - 1)
        def _():
            pltpu.sync_copy(acc_sh.at[pl.ds(r0, rpt)],
                            out_hbm.at[c, pl.ds(r0, rpt)])

        @pl.when(s == NS - 1)
        def _():
            pltpu.sync_copy(acc_sh.at[pl.ds((NS - 1) * rpt, rpt_last)],
                            out_hbm.at[c, pl.ds((NS - 1) * rpt, rpt_last)])

    return spmm(T, row2d, col2d, val2d, zeros)


def _sc_decode(z, src2d, dst2d, bias16, *, C=80):
    N, Z = z.shape
    E = src2d.shape[0] * C
    Et = E // NW
    nchunk = Et // C

    mesh = plsc.VectorSubcoreMesh(core_axis_name="c", subcore_axis_name="s",
                                  num_cores=NC, num_subcores=NS)

    @functools.partial(
        pl.kernel,
        out_type=jax.ShapeDtypeStruct((E // C, C), jnp.float32),
        mesh=mesh,
        compiler_params=pltpu.CompilerParams(use_tc_tiling_on_sc=False,
                                             needs_layout_passes=False),
        scratch_types=(
            [pltpu.VMEM((nchunk, C), jnp.int32)] * 2
            + [pltpu.VMEM((nchunk, C), jnp.float32)]
            + [pltpu.VMEM((L,), jnp.float32)]
            + [pltpu.VMEM((80 // L * L * L,), jnp.float32)]
            + [pltpu.VMEM((C, Z), jnp.float32)] * (2 * NBUF)
            + [pltpu.SemaphoreType.DMA] * (2 * NBUF)
        ),
    )
    def dec(z_hbm, src_hbm, dst_hbm, bias_hbm, out_hbm,
            srcv, dstv, obig, bvm, tbuf, *bufs_sems):
        zsb = bufs_sems[:NBUF]
        zdb = bufs_sems[NBUF:2 * NBUF]
        sem_a = bufs_sems[2 * NBUF:3 * NBUF]
        sem_b = bufs_sems[3 * NBUF:]
        c = lax.axis_index("c")
        s = lax.axis_index("s")
        wid = c * NS + s
        k0 = wid * nchunk

        pltpu.sync_copy(bias_hbm, bvm)
        b0 = bvm[pl.ds(0, L)][0]
        lanes = lax.iota(jnp.int32, L)
        pltpu.sync_copy(src_hbm.at[pl.ds(k0, nchunk)], srcv)
        pltpu.sync_copy(dst_hbm.at[pl.ds(k0, nchunk)], dstv)

        def gather(k, b):
            pltpu.async_copy(z_hbm.at[srcv.at[k]], zsb[b], sem_a[b])
            pltpu.async_copy(z_hbm.at[dstv.at[k]], zdb[b], sem_b[b])

        for b in range(NBUF):
            gather(b, b)

        def outer_body(g, carry):
            for b in range(NBUF):
                k = g * NBUF + b
                pltpu.make_async_copy(z_hbm.at[srcv.at[k]], zsb[b],
                                      sem_a[b]).wait()
                pltpu.make_async_copy(z_hbm.at[dstv.at[k]], zdb[b],
                                      sem_b[b]).wait()

                def group_body(gg, carry2):
                    for e2 in range(L):
                        e = gg * L + e2
                        acc = zsb[b][e, pl.ds(0, L)] * zdb[b][e, pl.ds(0, L)]
                        for j in range(1, Z // L):
                            acc = acc + (zsb[b][e, pl.ds(j * L, L)]
                                         * zdb[b][e, pl.ds(j * L, L)])
                        plsc.store_scatter(tbuf, [lanes * L + e2], acc)
                    tot = tbuf[pl.ds(0, L)]
                    for l in range(1, L):
                        tot = tot + tbuf[pl.ds(l * L, L)]
                    v = tot + b0
                    v = jnp.where(v != v, 0.0, v)
                    v = jnp.minimum(jnp.maximum(v, -20.0), 20.0)
                    obig[k, pl.ds(gg * L, L)] = v
                    return carry2

                lax.fori_loop(0, C // L, group_body, 0)
                gather(jnp.minimum(k + NBUF, nchunk - 1), b)
            return carry

        lax.fori_loop(0, nchunk // NBUF, outer_body, 0)
        for b in range(NBUF):
            pltpu.make_async_copy(z_hbm.at[srcv.at[nchunk - 1]], zsb[b],
                                  sem_a[b]).wait()
            pltpu.make_async_copy(z_hbm.at[dstv.at[nchunk - 1]], zdb[b],
                                  sem_b[b]).wait()
        pltpu.sync_copy(obig, out_hbm.at[pl.ds(k0, nchunk)])

    return dec(z, src2d, dst2d, bias16)


def kernel(X, adj_indices, adj_values, edge_index, W1, b1, W2, b2, dec_bias):
    N, D = X.shape
    H = W1.shape[0]
    Z = W2.shape[0]
    E = adj_values.shape[0]
    CS = 80
    CD = 80

    Ep = ((E + NW * CS - 1) // (NW * CS)) * (NW * CS)
    pad = Ep - E
    if pad:
        row = jnp.concatenate([adj_indices[0],
                               jnp.zeros((pad,), adj_indices.dtype)])
        col = jnp.concatenate([adj_indices[1],
                               jnp.zeros((pad,), adj_indices.dtype)])
        val = jnp.concatenate([adj_values,
                               jnp.zeros((pad,), adj_values.dtype)])
    else:
        row, col, val = adj_indices[0], adj_indices[1], adj_values
    row = row.reshape(Ep // CS, CS)
    col = col.reshape(Ep // CS, CS)
    val = val.reshape(Ep // CS, CS)

    xw = _tc_linear(X, W1, b1)
    hp = _sc_spmm(xw, row, col, val, jnp.zeros((N, H), jnp.float32),
                  C=CS, SBUF=2)
    hw = _tc_combine_act_linear(hp, W2, b2)
    zp = _sc_spmm(hw, row, col, val, jnp.zeros((N, Z), jnp.float32),
                  C=CS, SBUF=4)
    z = _tc_combine_clean(zp)
    bias16 = jnp.broadcast_to(dec_bias, (L,)).astype(jnp.float32)
    logits = _sc_decode(z, edge_index[0].reshape(E // CD, CD),
                        edge_index[1].reshape(E // CD, CD), bias16, C=CD)
    return logits.reshape(E)
